# baseline (speedup 1.0000x reference)
import functools
import jax
import jax.numpy as jnp
from jax import lax
from jax.experimental import pallas as pl
from jax.experimental.pallas import tpu as pltpu
from jax.experimental.pallas import tpu_sc as plsc

DIM = 64
LANES = 16
NUM_CORES = 2
NUM_SUBCORES = 16
NUM_WORKERS = NUM_CORES * NUM_SUBCORES

N_BAGS_CORE0 = 240
N_BAGS_CORE1 = 108


def _make_bag_kernel(n0, n1, chunk_tokens, bags_per_chunk, tokens_per_bag):
  max_bags = max(n0, n1)
  max_chunks = max_bags // bags_per_chunk
  pair_chunks = (n0 + n1) // bags_per_chunk
  mesh = plsc.VectorSubcoreMesh(core_axis_name="c", subcore_axis_name="s")

  @functools.partial(
      pl.kernel,
      mesh=mesh,
      out_type=jax.ShapeDtypeStruct(
          (NUM_WORKERS, max_bags, DIM), jnp.float32),
      scratch_types=[
          pltpu.VMEM((max_chunks, chunk_tokens), jnp.int32),
          pltpu.VMEM((chunk_tokens, DIM), jnp.float32),
          pltpu.VMEM((chunk_tokens, DIM), jnp.float32),
          pltpu.VMEM((chunk_tokens, DIM), jnp.float32),
          pltpu.VMEM((max_bags, DIM), jnp.float32),
          pltpu.SemaphoreType.DMA,
      ],
      compiler_params=pltpu.CompilerParams(
          needs_layout_passes=False, use_tc_tiling_on_sc=False),
  )
  def bag_kernel(table_hbm, x_hbm, out_hbm, x_v, rows0_v, rows1_v, rows2_v,
                 out_v, sem0):
    cid = lax.axis_index("c")
    sid = lax.axis_index("s")
    wid = sid * NUM_CORES + cid
    chunk_base = sid * pair_chunks + cid * (n0 // bags_per_chunk)
    n_me = jnp.where(cid == 0, n0 // bags_per_chunk, n1 // bags_per_chunk)
    pltpu.sync_copy(x_hbm.at[pl.ds(chunk_base, max_chunks)], x_v)

    def start_gather(c, buf, sem):
      pltpu.make_async_copy(table_hbm.at[x_v.at[c]], buf, sem).start()

    def wait_gather(buf, sem):
      pltpu.make_async_copy(table_hbm.at[x_v.at[0]], buf, sem).wait()

    def process(c, buf):
      for j in range(bags_per_chunk):
        base_t = j * tokens_per_bag
        zero = jnp.zeros((LANES,), jnp.float32)
        a0, a1, a2, a3 = zero, zero, zero, zero
        for t in range(tokens_per_bag):
          r = base_t + t
          a0 = a0 + buf[r, pl.ds(0, LANES)]
          a1 = a1 + buf[r, pl.ds(LANES, LANES)]
          a2 = a2 + buf[r, pl.ds(2 * LANES, LANES)]
          a3 = a3 + buf[r, pl.ds(3 * LANES, LANES)]

        i0 = x_v[c, pl.ds(base_t, LANES)]
        i1 = x_v[c, pl.ds(base_t + 16, LANES)]
        i2 = x_v[c, pl.ds(base_t + 32, LANES)]
        i3 = x_v[c, pl.ds(base_t + 34, LANES)]
        lane = lax.iota(jnp.int32, LANES)
        cnt = (plsc.all_reduce_population_count(i0 != 1)
               + plsc.all_reduce_population_count(i1 != 1)
               + plsc.all_reduce_population_count(i2 != 1)
               + plsc.all_reduce_population_count((i3 != 1) & (lane >= 14)))
        scale = 1.0 / cnt.astype(jnp.float32)
        b = c * bags_per_chunk + j
        out_v[b, pl.ds(0, LANES)] = a0 * scale
        out_v[b, pl.ds(LANES, LANES)] = a1 * scale
        out_v[b, pl.ds(2 * LANES, LANES)] = a2 * scale
        out_v[b, pl.ds(3 * LANES, LANES)] = a3 * scale

    start_gather(0, rows0_v, sem0)
    start_gather(1, rows1_v, sem0)
    bufs = (rows0_v, rows1_v, rows2_v)

    def triple_body(i, carry):
      c = 3 * i
      for k in range(3):
        ck = c + k

        @pl.when(ck + 2 < n_me)
        def _():
          start_gather(ck + 2, bufs[(k + 2) % 3], sem0)

        wait_gather(bufs[k], sem0)
        process(ck, bufs[k])
      return carry

    lax.fori_loop(0, n_me // 3, triple_body, 0)
    pltpu.sync_copy(out_v, out_hbm.at[wid])

  return bag_kernel


def kernel(x, table):
  b, l_tok = x.shape
  assert l_tok == 50 and table.shape[1] == DIM
  bags_per_chunk = 2
  chunk_tokens = bags_per_chunk * l_tok
  n0, n1 = N_BAGS_CORE0, N_BAGS_CORE1
  pair_bags = n0 + n1
  b_pad = NUM_SUBCORES * pair_bags
  assert b_pad >= b
  max_bags = max(n0, n1)
  max_chunks = max_bags // bags_per_chunk
  xp = jnp.pad(x.astype(jnp.int32), ((0, b_pad - b), (0, 0)),
               constant_values=1)
  xp = xp.reshape(-1, chunk_tokens)
  xp = jnp.concatenate(
      [xp, jnp.ones((max_chunks, chunk_tokens), jnp.int32)], axis=0)
  fn = _make_bag_kernel(n0, n1, chunk_tokens, bags_per_chunk, l_tok)
  out = fn(jnp.asarray(table, jnp.float32), xp)
  out = out.reshape(NUM_SUBCORES, NUM_CORES, max_bags, DIM)
  out = jnp.concatenate([out[:, 0, :n0], out[:, 1, :n1]], axis=1)
  return out.reshape(b_pad, DIM)[:b]


# speedup vs baseline: 1.4058x; 1.4058x over previous
import functools
import jax
import jax.numpy as jnp
from jax import lax
from jax.experimental import pallas as pl
from jax.experimental.pallas import tpu as pltpu
from jax.experimental.pallas import tpu_sc as plsc

DIM = 64
LANES = 16
NUM_CORES = 2
NUM_SUBCORES = 16
NUM_WORKERS = NUM_CORES * NUM_SUBCORES

N_BAGS_CORE0 = 240
N_BAGS_CORE1 = 104


def _make_bag_kernel(n0, n1, chunk_tokens, bags_per_chunk, tokens_per_bag):
  max_bags = max(n0, n1)
  max_chunks = max_bags // bags_per_chunk
  pair_chunks = (n0 + n1) // bags_per_chunk
  mesh = plsc.VectorSubcoreMesh(core_axis_name="c", subcore_axis_name="s")

  @functools.partial(
      pl.kernel,
      mesh=mesh,
      out_type=jax.ShapeDtypeStruct(
          (NUM_WORKERS, max_bags, DIM), jnp.float32),
      scratch_types=[
          pltpu.VMEM((max_chunks, chunk_tokens), jnp.int32),
          pltpu.VMEM((chunk_tokens, DIM), jnp.float32),
          pltpu.VMEM((chunk_tokens, DIM), jnp.float32),
          pltpu.VMEM((max_bags, DIM), jnp.float32),
          pltpu.SemaphoreType.DMA,
      ],
      compiler_params=pltpu.CompilerParams(
          needs_layout_passes=False, use_tc_tiling_on_sc=False),
  )
  def bag_kernel(table_hbm, x_hbm, out_hbm, x_v, rows0_v, rows1_v, out_v,
                 sem0):
    cid = lax.axis_index("c")
    sid = lax.axis_index("s")
    wid = sid * NUM_CORES + cid
    chunk_base = sid * pair_chunks + cid * (n0 // bags_per_chunk)
    n_me = jnp.where(cid == 0, n0 // bags_per_chunk, n1 // bags_per_chunk)
    pltpu.sync_copy(x_hbm.at[pl.ds(chunk_base, max_chunks)], x_v)

    def start_gather(c, buf, sem):
      pltpu.make_async_copy(table_hbm.at[x_v.at[c]], buf, sem).start()

    def wait_gather(buf, sem):
      pltpu.make_async_copy(table_hbm.at[x_v.at[0]], buf, sem).wait()

    def process(c, buf):
      for j in range(bags_per_chunk):
        base_t = j * tokens_per_bag
        zero = jnp.zeros((LANES,), jnp.float32)
        a0, a1, a2, a3 = zero, zero, zero, zero
        for t in range(tokens_per_bag):
          r = base_t + t
          a0 = a0 + buf[r, pl.ds(0, LANES)]
          a1 = a1 + buf[r, pl.ds(LANES, LANES)]
          a2 = a2 + buf[r, pl.ds(2 * LANES, LANES)]
          a3 = a3 + buf[r, pl.ds(3 * LANES, LANES)]

        i0 = x_v[c, pl.ds(base_t, LANES)]
        i1 = x_v[c, pl.ds(base_t + 16, LANES)]
        i2 = x_v[c, pl.ds(base_t + 32, LANES)]
        i3 = x_v[c, pl.ds(base_t + 34, LANES)]
        lane = lax.iota(jnp.int32, LANES)
        cnt = (plsc.all_reduce_population_count(i0 != 1)
               + plsc.all_reduce_population_count(i1 != 1)
               + plsc.all_reduce_population_count(i2 != 1)
               + plsc.all_reduce_population_count((i3 != 1) & (lane >= 14)))
        scale = 1.0 / cnt.astype(jnp.float32)
        b = c * bags_per_chunk + j
        out_v[b, pl.ds(0, LANES)] = a0 * scale
        out_v[b, pl.ds(LANES, LANES)] = a1 * scale
        out_v[b, pl.ds(2 * LANES, LANES)] = a2 * scale
        out_v[b, pl.ds(3 * LANES, LANES)] = a3 * scale

    start_gather(0, rows0_v, sem0)

    def pair_body(i, carry):
      c0 = 2 * i
      c1 = c0 + 1
      start_gather(c1, rows1_v, sem0)
      wait_gather(rows0_v, sem0)
      process(c0, rows0_v)

      @pl.when(c0 + 2 < n_me)
      def _():
        start_gather(c0 + 2, rows0_v, sem0)

      wait_gather(rows1_v, sem0)
      process(c1, rows1_v)
      return carry

    lax.fori_loop(0, n_me // 2, pair_body, 0)
    pltpu.sync_copy(out_v, out_hbm.at[wid])

  return bag_kernel


def kernel(x, table):
  b, l_tok = x.shape
  assert l_tok == 50 and table.shape[1] == DIM
  bags_per_chunk = 2
  chunk_tokens = bags_per_chunk * l_tok
  n0, n1 = N_BAGS_CORE0, N_BAGS_CORE1
  pair_bags = n0 + n1
  b_pad = NUM_SUBCORES * pair_bags
  assert b_pad >= b
  max_bags = max(n0, n1)
  max_chunks = max_bags // bags_per_chunk
  xp = jnp.pad(x.astype(jnp.int32), ((0, b_pad - b), (0, 0)),
               constant_values=1)
  xp = xp.reshape(-1, chunk_tokens)
  xp = jnp.concatenate(
      [xp, jnp.ones((max_chunks, chunk_tokens), jnp.int32)], axis=0)
  fn = _make_bag_kernel(n0, n1, chunk_tokens, bags_per_chunk, l_tok)
  out = fn(jnp.asarray(table, jnp.float32), xp)
  out = out.reshape(NUM_SUBCORES, NUM_CORES, max_bags, DIM)
  out = jnp.concatenate([out[:, 0, :n0], out[:, 1, :n1]], axis=1)
  return out.reshape(b_pad, DIM)[:b]
